# baseline (speedup 1.0000x reference)
import functools

import numpy as np
import jax
import jax.numpy as jnp
from jax import lax
from jax.experimental import pallas as pl
from jax.experimental.pallas import tpu as pltpu
from jax.experimental.pallas import tpu_sc as plsc

_FEAT = 26
_LEVELS = 100
_D = 2048
_B = 32
_T = 20
_W = 256
_NROWS = _T * _LEVELS


def _build_pack_mats():
    m1 = np.zeros((_D, _W), np.float32)
    for k in range(6):
        m1[np.arange(k * _W, (k + 1) * _W), np.arange(_W)] = float(1 << (4 * k))
    m2 = np.zeros((_D, _W), np.float32)
    for k in range(6, 8):
        m2[np.arange(k * _W, (k + 1) * _W), np.arange(_W)] = float(1 << (4 * (k - 6)))
    return m1, m2

_M1_NP, _M2_NP = _build_pack_mats()


def _prep_body(xt_ref, lvl_ref, id_ref, m1_ref, m2_ref,
               gidx_ref, lpack_ref, sidp_ref):
    raw = xt_ref[...] * np.float32(_LEVELS) - np.float32(1.0)
    q = raw.astype(jnp.int32)
    q = jnp.where(q < 0, q + _LEVELS, q)
    tt = lax.broadcasted_iota(jnp.int32, (_B, _FEAT, _T), 2)
    gidx_ref[...] = q + _LEVELS * tt

    m1 = m1_ref[...]
    m2 = m2_ref[...]

    def pack(bits_f32):
        p1 = jnp.dot(bits_f32.astype(jnp.bfloat16), m1,
                     preferred_element_type=jnp.float32)
        p2 = jnp.dot(bits_f32.astype(jnp.bfloat16), m2,
                     preferred_element_type=jnp.float32)
        return p1.astype(jnp.int32) + (p2.astype(jnp.int32) << 24)

    def packed_roll(base, t):
        if t == 0:
            return base
        u = base[:, _W - t:]
        rot = (u << 4) | ((u >> 28) & 15)
        return jnp.concatenate([rot, base[:, :_W - t]], axis=1)

    base = pack(lvl_ref[...])
    for t in range(_T):
        lpack_ref[pl.ds(t * _LEVELS, _LEVELS), :] = packed_roll(base, t)

    idb = pack(id_ref[...])
    acc = idb
    for t in range(1, _T):
        acc = acc ^ packed_roll(idb, t)
    sidp_ref[...] = acc


def _sc_body(lpack_hbm, gidx_hbm, sidp_hbm, out_hbm,
             idx_v, sidp_v, rows0_v, rows1_v, cnta_v, cntb_v, out_v,
             sem0, sem1):
    b = lax.axis_index("s") * 2 + lax.axis_index("c")
    pltpu.sync_copy(gidx_hbm.at[b], idx_v)
    pltpu.sync_copy(sidp_hbm, sidp_v)

    for s in range(2):
        for c in range(0, 128, 16):
            cnta_v[s, pl.ds(c, 16)] = jnp.zeros((16,), jnp.int32)
            cntb_v[s, pl.ds(c, 16)] = jnp.zeros((16,), jnp.int32)

    def start(f, rows_v, sem):
        pltpu.async_copy(lpack_hbm.at[idx_v.at[f]], rows_v, sem)

    def finish(f, rows_v, cnt_v, sem):
        pltpu.make_async_copy(lpack_hbm.at[idx_v.at[f]], rows_v, sem).wait()

        for s in range(2):
            for c in range(0, 128, 16):
                acc = rows_v[0, s, pl.ds(c, 16)]
                for t in range(1, _T):
                    acc = acc ^ rows_v[t, s, pl.ds(c, 16)]
                acc = acc ^ sidp_v[f, s, pl.ds(c, 16)]
                cnt_v[s, pl.ds(c, 16)] = cnt_v[s, pl.ds(c, 16)] + acc

    start(0, rows0_v, sem0)
    start(1, rows1_v, sem1)

    @pl.loop(0, _FEAT - 2, step=2)
    def _(f):
        finish(f, rows0_v, cnta_v, sem0)
        start(f + 2, rows0_v, sem0)
        finish(f + 1, rows1_v, cntb_v, sem1)
        start(f + 3, rows1_v, sem1)

    finish(_FEAT - 2, rows0_v, cnta_v, sem0)
    finish(_FEAT - 1, rows1_v, cntb_v, sem1)

    for k in range(8):
        for s in range(2):
            for c in range(0, 128, 16):
                va = (cnta_v[s, pl.ds(c, 16)] >> (4 * k)) & 15
                vb = (cntb_v[s, pl.ds(c, 16)] >> (4 * k)) & 15
                out_v[2 * k + s, pl.ds(c, 16)] = jnp.where(
                    va + vb > 13, np.float32(1.0), np.float32(-1.0))

    pltpu.sync_copy(out_v, out_hbm.at[b])


def kernel(x, level_hvs, id_hvs):
    xt = jnp.swapaxes(x, 1, 2)
    m1 = jnp.asarray(_M1_NP, jnp.bfloat16)
    m2 = jnp.asarray(_M2_NP, jnp.bfloat16)

    gidx, lpack, sidp = pl.pallas_call(
        _prep_body,
        out_shape=[
            jax.ShapeDtypeStruct((_B, _FEAT, _T), jnp.int32),
            jax.ShapeDtypeStruct((_NROWS, _W), jnp.int32),
            jax.ShapeDtypeStruct((_FEAT, _W), jnp.int32),
        ],
    )(xt, level_hvs, id_hvs, m1, m2)

    mesh = plsc.VectorSubcoreMesh(core_axis_name="c", subcore_axis_name="s")
    sc_accumulate = functools.partial(
        pl.kernel,
        out_type=jax.ShapeDtypeStruct((_B, 16, 128), jnp.float32),
        mesh=mesh,
        scratch_types=[
            pltpu.VMEM((_FEAT, _T), jnp.int32),
            pltpu.VMEM((_FEAT, 2, 128), jnp.int32),
            pltpu.VMEM((_T, 2, 128), jnp.int32),
            pltpu.VMEM((_T, 2, 128), jnp.int32),
            pltpu.VMEM((2, 128), jnp.int32),
            pltpu.VMEM((2, 128), jnp.int32),
            pltpu.VMEM((16, 128), jnp.float32),
            pltpu.SemaphoreType.DMA,
            pltpu.SemaphoreType.DMA,
        ],
    )(_sc_body)
    out = sc_accumulate(lpack.reshape(_NROWS, 2, 128), gidx,
                        sidp.reshape(_FEAT, 2, 128))
    return out.reshape(_B, _D)


# speedup vs baseline: 1.1447x; 1.1447x over previous
import functools

import numpy as np
import jax
import jax.numpy as jnp
from jax import lax
from jax.experimental import pallas as pl
from jax.experimental.pallas import tpu as pltpu
from jax.experimental.pallas import tpu_sc as plsc

_FEAT = 26
_LEVELS = 100
_D = 2048
_B = 32
_T = 20
_W = 256
_NROWS = _T * _LEVELS


def _build_pack_mats():
    m1 = np.zeros((_D, _W), np.float32)
    for k in range(6):
        m1[np.arange(k * _W, (k + 1) * _W), np.arange(_W)] = float(1 << (4 * k))
    m2 = np.zeros((_D, _W), np.float32)
    for k in range(6, 8):
        m2[np.arange(k * _W, (k + 1) * _W), np.arange(_W)] = float(1 << (4 * (k - 6)))
    return m1, m2

_M1_NP, _M2_NP = _build_pack_mats()


def _prep_body(xt_ref, lvl_ref, id_ref, m1_ref, m2_ref,
               gidx_ref, lpack_ref, sidp_ref):
    raw = xt_ref[...] * np.float32(_LEVELS) - np.float32(1.0)
    q = raw.astype(jnp.int32)
    q = jnp.where(q < 0, q + _LEVELS, q)
    tt = lax.broadcasted_iota(jnp.int32, (_B, _FEAT, _T), 2)
    gidx_ref[...] = q + _LEVELS * tt

    m1 = m1_ref[...]
    m2 = m2_ref[...]

    def pack(bits_f32):
        p1 = jnp.dot(bits_f32.astype(jnp.bfloat16), m1,
                     preferred_element_type=jnp.float32)
        p2 = jnp.dot(bits_f32.astype(jnp.bfloat16), m2,
                     preferred_element_type=jnp.float32)
        return p1.astype(jnp.int32) + (p2.astype(jnp.int32) << 24)

    def packed_roll(base, t):
        if t == 0:
            return base
        u = base[:, _W - t:]
        rot = (u << 4) | ((u >> 28) & 15)
        return jnp.concatenate([rot, base[:, :_W - t]], axis=1)

    base = pack(lvl_ref[...])
    for t in range(_T):
        lpack_ref[pl.ds(t * _LEVELS, _LEVELS), :] = packed_roll(base, t)

    idb = pack(id_ref[...])
    acc = idb
    for t in range(1, _T):
        acc = acc ^ packed_roll(idb, t)
    sidp_ref[...] = acc


def _sc_body(lpack_hbm, gidx_hbm, sidp_hbm, out_hbm,
             idx_v, sidp_v, rows0_v, rows1_v, cnta_v, cntb_v, out_v, ltab_sh,
             sem0, sem1):
    sid = lax.axis_index("s")
    b = sid * 2 + lax.axis_index("c")
    pltpu.sync_copy(gidx_hbm.at[b], idx_v)
    pltpu.sync_copy(sidp_hbm, sidp_v)

    rows_per_sub = _NROWS // 16
    pltpu.sync_copy(lpack_hbm.at[pl.ds(sid * rows_per_sub, rows_per_sub)],
                    ltab_sh.at[pl.ds(sid * rows_per_sub, rows_per_sub)])
    plsc.subcore_barrier()

    for s in range(2):
        for c in range(0, 128, 16):
            cnta_v[s, pl.ds(c, 16)] = jnp.zeros((16,), jnp.int32)
            cntb_v[s, pl.ds(c, 16)] = jnp.zeros((16,), jnp.int32)

    def start(f, rows_v, sem):
        pltpu.async_copy(ltab_sh.at[idx_v.at[f]], rows_v, sem)

    def finish(f, rows_v, cnt_v, sem):
        pltpu.make_async_copy(ltab_sh.at[idx_v.at[f]], rows_v, sem).wait()

        @pl.loop(0, 2)
        def _(s):
            @pl.loop(0, 128, step=16)
            def _(c):
                acc = rows_v[0, s, pl.ds(c, 16)]
                for t in range(1, _T):
                    acc = acc ^ rows_v[t, s, pl.ds(c, 16)]
                acc = acc ^ sidp_v[f, s, pl.ds(c, 16)]
                cnt_v[s, pl.ds(c, 16)] = cnt_v[s, pl.ds(c, 16)] + acc

    start(0, rows0_v, sem0)
    start(1, rows1_v, sem1)

    @pl.loop(0, _FEAT - 2, step=2)
    def _(f):
        finish(f, rows0_v, cnta_v, sem0)
        start(f + 2, rows0_v, sem0)
        finish(f + 1, rows1_v, cntb_v, sem1)
        start(f + 3, rows1_v, sem1)

    finish(_FEAT - 2, rows0_v, cnta_v, sem0)
    finish(_FEAT - 1, rows1_v, cntb_v, sem1)

    for k in range(8):
        for s in range(2):
            for c in range(0, 128, 16):
                va = (cnta_v[s, pl.ds(c, 16)] >> (4 * k)) & 15
                vb = (cntb_v[s, pl.ds(c, 16)] >> (4 * k)) & 15
                out_v[2 * k + s, pl.ds(c, 16)] = jnp.where(
                    va + vb > 13, np.float32(1.0), np.float32(-1.0))

    pltpu.sync_copy(out_v, out_hbm.at[b])


def kernel(x, level_hvs, id_hvs):
    xt = jnp.swapaxes(x, 1, 2)
    m1 = jnp.asarray(_M1_NP, jnp.bfloat16)
    m2 = jnp.asarray(_M2_NP, jnp.bfloat16)

    gidx, lpack, sidp = pl.pallas_call(
        _prep_body,
        out_shape=[
            jax.ShapeDtypeStruct((_B, _FEAT, _T), jnp.int32),
            jax.ShapeDtypeStruct((_NROWS, _W), jnp.int32),
            jax.ShapeDtypeStruct((_FEAT, _W), jnp.int32),
        ],
    )(xt, level_hvs, id_hvs, m1, m2)

    mesh = plsc.VectorSubcoreMesh(core_axis_name="c", subcore_axis_name="s")
    sc_accumulate = functools.partial(
        pl.kernel,
        out_type=jax.ShapeDtypeStruct((_B, 16, 128), jnp.float32),
        mesh=mesh,
        scratch_types=[
            pltpu.VMEM((_FEAT, _T), jnp.int32),
            pltpu.VMEM((_FEAT, 2, 128), jnp.int32),
            pltpu.VMEM((_T, 2, 128), jnp.int32),
            pltpu.VMEM((_T, 2, 128), jnp.int32),
            pltpu.VMEM((2, 128), jnp.int32),
            pltpu.VMEM((2, 128), jnp.int32),
            pltpu.VMEM((16, 128), jnp.float32),
            pltpu.VMEM_SHARED((_NROWS, 2, 128), jnp.int32),
            pltpu.SemaphoreType.DMA,
            pltpu.SemaphoreType.DMA,
        ],
    )(_sc_body)
    out = sc_accumulate(lpack.reshape(_NROWS, 2, 128), gidx,
                        sidp.reshape(_FEAT, 2, 128))
    return out.reshape(_B, _D)
